# baseline (speedup 1.0000x reference)
import functools

import jax
import jax.numpy as jnp
from jax import lax
from jax.experimental import pallas as pl
from jax.experimental.pallas import tpu as pltpu
from jax.experimental.pallas import tpu_sc as plsc

N = 10000
E = 320000
D = 128

NC = 2
NS = 16
CH = 128
K = 160
SLAB = 40
NBUF = 4
E_PAD = NS * K * CH
N_PAD = 10240
NH = N_PAD // NC
ST2 = NH // NS
DW = 8
SENT = 2 ** 30
LANES = 16


def _sc_aggregate(x, src_r, dst_r, za, zd, ones):
    mesh = plsc.VectorSubcoreMesh(core_axis_name="c", subcore_axis_name="s")

    @functools.partial(
        pl.kernel,
        out_type=[
            jax.ShapeDtypeStruct((N_PAD, D), jnp.float32),
            jax.ShapeDtypeStruct((N_PAD, DW), jnp.float32),
        ],
        mesh=mesh,
        scratch_types=[
            pltpu.VMEM((SLAB, CH), jnp.int32),
            pltpu.VMEM((SLAB, CH), jnp.int32),
            pltpu.VMEM((NBUF, CH, D), jnp.float32),
            pltpu.VMEM((CH, DW), jnp.float32),
            pltpu.VMEM((CH, DW), jnp.float32),
            pltpu.VMEM_SHARED((NH, D), jnp.float32),
            pltpu.VMEM_SHARED((NH, DW), jnp.float32),
            pltpu.SemaphoreType.DMA((NBUF,)),
            pltpu.SemaphoreType.DMA((NBUF,)),
            pltpu.SemaphoreType.DMA,
        ],
        compiler_params=pltpu.CompilerParams(use_tc_tiling_on_sc=False),
    )
    def run(x_hbm, src_hbm, dst_hbm, za_hbm, zd_hbm, ones_hbm,
            agg_out, deg_out,
            src_v, dst_v, rows_v, ones_v, degst_v, agg_sh, deg_sh,
            gsem, ssem, dsem):
        c = lax.axis_index("c")
        s = lax.axis_index("s")
        lo = c * NH
        base = s * ST2
        stripe_parts = ((0, 128), (128, 128), (256, 64))

        pltpu.sync_copy(za_hbm, rows_v.at[0])
        pltpu.sync_copy(zd_hbm, degst_v)
        for o, ln in stripe_parts:
            pltpu.sync_copy(rows_v.at[0].at[pl.ds(0, ln)],
                            agg_sh.at[pl.ds(base + o, ln)])
            pltpu.sync_copy(degst_v.at[pl.ds(0, ln)],
                            deg_sh.at[pl.ds(base + o, ln)])
        pltpu.sync_copy(ones_hbm, ones_v)

        plsc.subcore_barrier()

        for half in range(K // SLAB):
            hb = half * SLAB
            pltpu.sync_copy(src_hbm.at[s, pl.ds(hb, SLAB)], src_v)
            pltpu.sync_copy(dst_hbm.at[s, pl.ds(hb, SLAB)], dst_v)

            def filt(k, carry):
                for m in range(CH // LANES):
                    sl = pl.ds(m * LANES, LANES)
                    dvec = dst_v[k, sl]
                    svec = src_v[k, sl]
                    own = (dvec >= lo) & (dvec < lo + NH)
                    src_v[k, sl] = jnp.where(own, svec, SENT)
                    dst_v[k, sl] = jnp.where(own, dvec - lo, SENT)
                return carry

            lax.fori_loop(0, SLAB, filt, 0)

            def gidx(ck):
                return plsc.Indices(src_v.at[ck], ignored_value=SENT)

            def sidx(ck):
                return plsc.Indices(dst_v.at[ck], ignored_value=SENT)

            for b in range(NBUF):
                pltpu.async_copy(
                    x_hbm.at[gidx(b)], rows_v.at[b], gsem.at[b])

            def step(i, carry):
                for b in range(NBUF):
                    ck = i * NBUF + b
                    pltpu.make_async_copy(
                        x_hbm.at[gidx(ck)], rows_v.at[b], gsem.at[b]).wait()
                    pltpu.async_copy(
                        rows_v.at[b], agg_sh.at[sidx(ck)], ssem.at[b],
                        add=True)
                    pltpu.async_copy(
                        ones_v, deg_sh.at[sidx(ck)], dsem, add=True)

                    @pl.when(ck + NBUF < SLAB)
                    def _():
                        pltpu.make_async_copy(
                            rows_v.at[b], agg_sh.at[sidx(ck)],
                            ssem.at[b]).wait()
                        pltpu.async_copy(
                            x_hbm.at[gidx(ck + NBUF)], rows_v.at[b],
                            gsem.at[b])

                return carry

            lax.fori_loop(0, SLAB // NBUF, step, 0)

            for b in range(NBUF):
                pltpu.make_async_copy(
                    rows_v.at[b], agg_sh.at[sidx(0)], ssem.at[b]).wait()

            def dwait(i, carry):
                pltpu.make_async_copy(
                    ones_v, deg_sh.at[sidx(0)], dsem).wait()
                return carry

            lax.fori_loop(0, SLAB, dwait, 0)

        plsc.subcore_barrier()

        for o, ln in stripe_parts:
            pltpu.sync_copy(agg_sh.at[pl.ds(base + o, ln)],
                            rows_v.at[0].at[pl.ds(0, ln)])
            pltpu.sync_copy(rows_v.at[0].at[pl.ds(0, ln)],
                            agg_out.at[pl.ds(lo + base + o, ln)])
            pltpu.sync_copy(deg_sh.at[pl.ds(base + o, ln)],
                            degst_v.at[pl.ds(0, ln)])
            pltpu.sync_copy(degst_v.at[pl.ds(0, ln)],
                            deg_out.at[pl.ds(lo + base + o, ln)])

    return run(x, src_r, dst_r, za, zd, ones)


def _tc_body(x_ref, a_ref, d_ref, ws_ref, wn_ref, bs_ref, bn_ref, o_ref):
    xb = x_ref[...]
    deg = jnp.maximum(d_ref[...], 1.0)
    h = (jnp.dot(xb, ws_ref[...], preferred_element_type=jnp.float32)
         + jnp.dot(a_ref[...] / deg, wn_ref[...],
                   preferred_element_type=jnp.float32)
         + bs_ref[...] + bn_ref[...])
    o_ref[...] = jnp.maximum(h, 0.0)


def kernel(x, edge_index, W_self, b_self, W_neigh, b_neigh):
    src = edge_index[0].astype(jnp.int32)
    dst = edge_index[1].astype(jnp.int32)
    pad_e = E_PAD - E
    src = jnp.concatenate([src, jnp.zeros((pad_e,), jnp.int32)])
    pad_dst = N + (jnp.arange(pad_e, dtype=jnp.int32) % (N_PAD - N))
    dst = jnp.concatenate([dst, pad_dst])
    src_r = src.reshape(NS, K, CH)
    dst_r = dst.reshape(NS, K, CH)
    za = jnp.zeros((CH, D), jnp.float32)
    zd = jnp.zeros((CH, DW), jnp.float32)
    ones = jnp.ones((CH, DW), jnp.float32)

    agg_p, deg_p = _sc_aggregate(x, src_r, dst_r, za, zd, ones)
    deg1 = deg_p[:, :1]

    R = 1000
    h = pl.pallas_call(
        _tc_body,
        grid=(N // R,),
        in_specs=[
            pl.BlockSpec((R, D), lambda i: (i, 0)),
            pl.BlockSpec((R, D), lambda i: (i, 0)),
            pl.BlockSpec((R, 1), lambda i: (i, 0)),
            pl.BlockSpec((D, D), lambda i: (0, 0)),
            pl.BlockSpec((D, D), lambda i: (0, 0)),
            pl.BlockSpec((1, D), lambda i: (0, 0)),
            pl.BlockSpec((1, D), lambda i: (0, 0)),
        ],
        out_specs=pl.BlockSpec((R, D), lambda i: (i, 0)),
        out_shape=jax.ShapeDtypeStruct((N, D), jnp.float32),
    )(x, agg_p, deg1, W_self.T, W_neigh.T,
      b_self.reshape(1, D), b_neigh.reshape(1, D))
    return h


# speedup vs baseline: 1.4575x; 1.4575x over previous
import functools

import jax
import jax.numpy as jnp
from jax import lax
from jax.experimental import pallas as pl
from jax.experimental.pallas import tpu as pltpu
from jax.experimental.pallas import tpu_sc as plsc

N = 10000
E = 320000
D = 128
DH = D // 2

NC = 2
NS = 16
CH = 128
K = 160
NBUF = 5
E_PAD = NS * K * CH
N_PAD = 10240
STRIPE = N_PAD // NS
DW = 8


def _sc_aggregate(x, src_r, dst_r, za, zd, ones):
    mesh = plsc.VectorSubcoreMesh(core_axis_name="c", subcore_axis_name="s")

    @functools.partial(
        pl.kernel,
        out_type=[
            jax.ShapeDtypeStruct((N_PAD, D), jnp.float32),
            jax.ShapeDtypeStruct((NC, N_PAD, DW), jnp.float32),
        ],
        mesh=mesh,
        scratch_types=[
            pltpu.VMEM((K, CH), jnp.int32),
            pltpu.VMEM((K, CH), jnp.int32),
            pltpu.VMEM((NBUF, CH, DH), jnp.float32),
            pltpu.VMEM((CH, DW), jnp.float32),
            pltpu.VMEM((CH, DW), jnp.float32),
            pltpu.VMEM_SHARED((N_PAD, DH), jnp.float32),
            pltpu.VMEM_SHARED((N_PAD, DW), jnp.float32),
            pltpu.SemaphoreType.DMA((NBUF,)),
            pltpu.SemaphoreType.DMA((NBUF,)),
            pltpu.SemaphoreType.DMA,
        ],
        compiler_params=pltpu.CompilerParams(use_tc_tiling_on_sc=False),
    )
    def run(x_hbm, src_hbm, dst_hbm, za_hbm, zd_hbm, ones_hbm,
            agg_out, deg_out,
            src_v, dst_v, rows_v, ones_v, degst_v, agg_sh, deg_sh,
            gsem, ssem, dsem):
        c = lax.axis_index("c")
        s = lax.axis_index("s")
        pltpu.sync_copy(za_hbm, rows_v.at[0])
        pltpu.sync_copy(zd_hbm, degst_v)
        for t in range(STRIPE // CH):
            off = s * STRIPE + t * CH
            pltpu.sync_copy(rows_v.at[0], agg_sh.at[pl.ds(off, CH)])
            pltpu.sync_copy(degst_v, deg_sh.at[pl.ds(off, CH)])
        pltpu.sync_copy(ones_hbm, ones_v)

        pltpu.sync_copy(src_hbm.at[c, s], src_v)
        pltpu.sync_copy(dst_hbm.at[s], dst_v)

        plsc.subcore_barrier()

        for b in range(NBUF):
            pltpu.async_copy(
                x_hbm.at[src_v.at[b]], rows_v.at[b], gsem.at[b])

        def step(i, carry):
            for b in range(NBUF):
                ck = i * NBUF + b
                pltpu.make_async_copy(
                    x_hbm.at[src_v.at[ck]], rows_v.at[b], gsem.at[b]).wait()
                pltpu.async_copy(
                    rows_v.at[b], agg_sh.at[dst_v.at[ck]], ssem.at[b],
                    add=True)

                @pl.when(ck % NC == c)
                def _():
                    pltpu.async_copy(
                        ones_v, deg_sh.at[dst_v.at[ck]], dsem, add=True)

                @pl.when(ck + NBUF < K)
                def _():
                    pltpu.make_async_copy(
                        rows_v.at[b], agg_sh.at[dst_v.at[ck]],
                        ssem.at[b]).wait()
                    pltpu.async_copy(
                        x_hbm.at[src_v.at[ck + NBUF]], rows_v.at[b],
                        gsem.at[b])

            return carry

        lax.fori_loop(0, K // NBUF, step, 0)

        for b in range(NBUF):
            pltpu.make_async_copy(
                rows_v.at[b], agg_sh.at[dst_v.at[0]], ssem.at[b]).wait()

        def dwait(i, carry):
            pltpu.make_async_copy(
                ones_v, deg_sh.at[dst_v.at[0]], dsem).wait()
            return carry

        lax.fori_loop(0, K // NC, dwait, 0)

        plsc.subcore_barrier()

        for t in range(STRIPE // CH):
            off = s * STRIPE + t * CH
            pltpu.sync_copy(agg_sh.at[pl.ds(off, CH)], rows_v.at[0])
            pltpu.sync_copy(
                rows_v.at[0],
                agg_out.at[pl.ds(off, CH), pl.ds(c * DH, DH)])
            pltpu.sync_copy(deg_sh.at[pl.ds(off, CH)], degst_v)
            pltpu.sync_copy(degst_v, deg_out.at[c, pl.ds(off, CH)])

    return run(x, src_r, dst_r, za, zd, ones)


def _tc_body(x_ref, a_ref, d_ref, ws_ref, wn_ref, bs_ref, bn_ref, o_ref):
    xb = x_ref[...]
    deg = jnp.maximum(d_ref[0] + d_ref[1], 1.0)
    h = (jnp.dot(xb, ws_ref[...], preferred_element_type=jnp.float32)
         + jnp.dot(a_ref[...] / deg, wn_ref[...],
                   preferred_element_type=jnp.float32)
         + bs_ref[...] + bn_ref[...])
    o_ref[...] = jnp.maximum(h, 0.0)


def kernel(x, edge_index, W_self, b_self, W_neigh, b_neigh):
    src = edge_index[0].astype(jnp.int32)
    dst = edge_index[1].astype(jnp.int32)
    pad_e = E_PAD - E
    src = jnp.concatenate([src, jnp.zeros((pad_e,), jnp.int32)])
    pad_dst = N + (jnp.arange(pad_e, dtype=jnp.int32) % (N_PAD - N))
    dst = jnp.concatenate([dst, pad_dst])
    src_r = src.reshape(NS, K, CH)
    dst_r = dst.reshape(NS, K, CH)
    xh = jnp.concatenate([x[:, :DH], x[:, DH:]], axis=0)
    src2 = jnp.stack([src_r, src_r + N])
    za = jnp.zeros((CH, DH), jnp.float32)
    zd = jnp.zeros((CH, DW), jnp.float32)
    ones = jnp.ones((CH, DW), jnp.float32)

    agg_p, deg_p = _sc_aggregate(xh, src2, dst_r, za, zd, ones)
    deg1 = deg_p[:, :, :1]

    R = 1000
    h = pl.pallas_call(
        _tc_body,
        grid=(N // R,),
        in_specs=[
            pl.BlockSpec((R, D), lambda i: (i, 0)),
            pl.BlockSpec((R, D), lambda i: (i, 0)),
            pl.BlockSpec((2, R, 1), lambda i: (0, i, 0)),
            pl.BlockSpec((D, D), lambda i: (0, 0)),
            pl.BlockSpec((D, D), lambda i: (0, 0)),
            pl.BlockSpec((1, D), lambda i: (0, 0)),
            pl.BlockSpec((1, D), lambda i: (0, 0)),
        ],
        out_specs=pl.BlockSpec((R, D), lambda i: (i, 0)),
        out_shape=jax.ShapeDtypeStruct((N, D), jnp.float32),
    )(x, agg_p, deg1, W_self.T, W_neigh.T,
      b_self.reshape(1, D), b_neigh.reshape(1, D))
    return h
